# Initial kernel scaffold; baseline (speedup 1.0000x reference)
import jax
import jax.numpy as jnp
from jax.experimental import pallas as pl


def kernel(input_ids, offsets, edge_index, edge_ppi, edge_self, emb_table, input_bias, W1, b1, W2, b2, W_out, b_out):
    raise NotImplementedError("write your pallas kernel here")


# speedup vs baseline: 37.1892x; 37.1892x over previous
import functools

import jax
import jax.numpy as jnp
from jax import lax
from jax.experimental import pallas as pl
from jax.experimental.pallas import tpu as pltpu
from jax.experimental.pallas import tpu_sc as plsc

_NC = 2
_NS = 16
_NW = _NC * _NS

_CH = 64
_CE = 128


def _sc_mesh():
    return plsc.VectorSubcoreMesh(
        core_axis_name="c", subcore_axis_name="s",
        num_cores=_NC, num_subcores=_NS)


def _make_bag(N, D, NPT, OB):
    npad = _NW * NPT

    @functools.partial(
        pl.kernel,
        out_type=jax.ShapeDtypeStruct((npad, D), jnp.float32),
        mesh=_sc_mesh(),
        scratch_types=[
            pltpu.VMEM((OB,), jnp.int32),
            pltpu.VMEM((_CH + 8,), jnp.int32),
            pltpu.VMEM((_CH + 8, D), jnp.float32),
            pltpu.VMEM((NPT, D), jnp.float32),
            pltpu.VMEM((D,), jnp.float32),
            pltpu.SemaphoreType.DMA,
        ],
    )
    def bag(ids_hbm, off_hbm, emb_hbm, bias_hbm, h_hbm,
            offv, idxv, rowsv, outv, biasv, sem):
        wid = lax.axis_index("s") * _NC + lax.axis_index("c")
        n0 = pl.multiple_of(wid * NPT, 8)
        a0 = n0
        pltpu.sync_copy(off_hbm.at[pl.ds(a0, OB)], offv)
        pltpu.sync_copy(bias_hbm, biasv)
        nn = jnp.minimum(N - n0, NPT)

        def node_body(i, carry):
            n = n0 + i
            offpair = offv[pl.ds(n - a0, 16)]
            j0 = offpair[0]
            j1 = offpair[1]
            k = j1 - j0
            nch = (k + _CH - 1) // _CH

            def chunk(c, acc):
                start = j0 + c * _CH
                a = pl.multiple_of((start // 8) * 8, 8)
                ofs = start - a
                pltpu.sync_copy(ids_hbm.at[pl.ds(a, _CH + 8)], idxv)
                pltpu.async_copy(emb_hbm.at[idxv], rowsv, sem).wait()
                valid = jnp.minimum(k - c * _CH, _CH)

                def row(r, acc2):
                    return tuple(acc2[u] + rowsv[r, pl.ds(u * 16, 16)]
                                 for u in range(8))

                return lax.fori_loop(ofs, ofs + valid, row, acc)

            zero = jnp.zeros((16,), jnp.float32)
            acc = lax.fori_loop(0, nch, chunk, (zero,) * 8)
            for u in range(8):
                outv[i, pl.ds(u * 16, 16)] = jnp.maximum(
                    acc[u] + biasv[pl.ds(u * 16, 16)], 0.0)
            return carry

        lax.fori_loop(0, nn, node_body, 0)
        pltpu.sync_copy(outv, h_hbm.at[pl.ds(n0, NPT), :])

    return bag


def _make_edge(N, E, D):
    ept = E // _NS
    fpt = ((N // _NS + 127) // 128) * 128
    zb = fpt // 5
    npad = _NS * fpt

    @functools.partial(
        pl.kernel,
        out_type=jax.ShapeDtypeStruct((2, npad, D), jnp.float32),
        mesh=_sc_mesh(),
        scratch_types=[
            pltpu.VMEM((_CE,), jnp.int32),
            pltpu.VMEM((_CE,), jnp.int32),
            pltpu.VMEM((_CE + 16,), jnp.float32),
            pltpu.VMEM((_CE, D), jnp.float32),
            pltpu.VMEM((zb, D), jnp.float32),
            pltpu.VMEM_SHARED((npad, D), jnp.float32),
            pltpu.SemaphoreType.DMA,
        ],
    )
    def edge(src_hbm, dst_hbm, ew_hbm, h_hbm, o_hbm,
             srcv, dstv, wv, rowsv, zv, acc, sem):
        cid = lax.axis_index("c")
        sid = lax.axis_index("s")

        zero = jnp.zeros((16,), jnp.float32)

        def zrow(r, carry):
            for u in range(8):
                zv[r, pl.ds(u * 16, 16)] = zero
            return carry

        lax.fori_loop(0, zb, zrow, 0)
        for q in range(5):
            pltpu.sync_copy(
                zv, acc.at[pl.ds(pl.multiple_of(sid * fpt + q * zb, 8),
                                 zb), :])
        plsc.subcore_barrier()

        e0 = sid * ept

        def chunk(c, carry):
            eb = e0 + c * _CE
            pltpu.sync_copy(src_hbm.at[pl.ds(eb, _CE)], srcv)
            pltpu.sync_copy(dst_hbm.at[pl.ds(eb, _CE)], dstv)
            pltpu.sync_copy(
                ew_hbm.at[pl.ds(pl.multiple_of(cid * E + eb, 128), _CE)],
                wv.at[pl.ds(0, _CE)])
            pltpu.async_copy(h_hbm.at[srcv], rowsv, sem).wait()

            def srow(e, c2):
                w = wv[pl.ds(e, 16)][0]
                for u in range(8):
                    rowsv[e, pl.ds(u * 16, 16)] = (
                        rowsv[e, pl.ds(u * 16, 16)] * w)
                return c2

            lax.fori_loop(0, _CE, srow, 0)
            pltpu.sync_copy(rowsv, acc.at[dstv], add=True)
            return carry

        lax.fori_loop(0, ept // _CE, chunk, 0)
        plsc.subcore_barrier()
        f0 = pl.multiple_of(sid * fpt, 8)
        pltpu.sync_copy(acc.at[pl.ds(f0, fpt), :],
                        o_hbm.at[cid, pl.ds(f0, fpt), :])

    return edge


def _dense(res, ppi, W, b):
    N, D = ppi.shape
    BN = 1000

    def body(res_ref, ppi_ref, w_ref, b_ref, out_ref):
        y = lax.dot_general(ppi_ref[...], w_ref[...],
                            (((1,), (1,)), ((), ())),
                            preferred_element_type=jnp.float32)
        out_ref[...] = jnp.maximum(y + b_ref[...] + res_ref[...], 0.0)

    return pl.pallas_call(
        body,
        grid=(N // BN,),
        in_specs=[
            pl.BlockSpec((BN, D), lambda i: (i, 0)),
            pl.BlockSpec((BN, D), lambda i: (i, 0)),
            pl.BlockSpec((D, D), lambda i: (0, 0)),
            pl.BlockSpec((1, D), lambda i: (0, 0)),
        ],
        out_specs=pl.BlockSpec((BN, D), lambda i: (i, 0)),
        out_shape=jax.ShapeDtypeStruct((N, D), jnp.float32),
    )(res, ppi, W, b.reshape(1, D))


def _proj(h, W_out, b_out):
    N, D = h.shape
    C = W_out.shape[0]
    BN = 1000

    def body(h_ref, w_ref, b_ref, out_ref):
        y = lax.dot_general(h_ref[...], w_ref[...],
                            (((1,), (1,)), ((), ())),
                            preferred_element_type=jnp.float32)
        out_ref[...] = y + b_ref[...]

    return pl.pallas_call(
        body,
        grid=(N // BN,),
        in_specs=[
            pl.BlockSpec((BN, D), lambda i: (i, 0)),
            pl.BlockSpec((C, D), lambda i: (0, 0)),
            pl.BlockSpec((1, C), lambda i: (0, 0)),
        ],
        out_specs=pl.BlockSpec((BN, C), lambda i: (i, 0)),
        out_shape=jax.ShapeDtypeStruct((N, C), jnp.float32),
    )(h, W_out, b_out.reshape(1, C))


def kernel(input_ids, offsets, edge_index, edge_ppi, edge_self,
           emb_table, input_bias, W1, b1, W2, b2, W_out, b_out):
    L = input_ids.shape[0]
    N = offsets.shape[0] - 1
    V, D = emb_table.shape
    E = edge_index.shape[1]

    npt = ((-(-N // _NW) + 7) // 8) * 8
    ob = ((npt + 32) // 8) * 8

    ids_pad = jnp.pad(input_ids.astype(jnp.int32), (0, _CH + 16))
    off_pad = jnp.pad(offsets.astype(jnp.int32), (0, ob),
                      constant_values=L)

    h = _make_bag(N, D, npt, ob)(ids_pad, off_pad, emb_table,
                                 input_bias)[:N]

    ep = -(-E // (_NS * _CE)) * _NS * _CE
    pad_e = ep - E
    src = jnp.pad(edge_index[0].astype(jnp.int32), (0, pad_e))
    dst = jnp.pad(edge_index[1].astype(jnp.int32), (0, pad_e),
                  constant_values=N)
    ew = jnp.concatenate([jnp.pad(edge_self, (0, pad_e)),
                          jnp.pad(edge_ppi, (0, pad_e))])

    edge_fn = _make_edge(N, ep, D)
    for W, b in ((W1, b1), (W2, b2)):
        o = edge_fn(src, dst, ew, h)
        h = _dense(o[0, :N], o[1, :N], W, b)

    return _proj(h, W_out, b_out)


# speedup vs baseline: 61.4600x; 1.6526x over previous
import functools

import jax
import jax.numpy as jnp
from jax import lax
from jax.experimental import pallas as pl
from jax.experimental.pallas import tpu as pltpu
from jax.experimental.pallas import tpu_sc as plsc

_NC = 2
_NS = 16
_NW = _NC * _NS

_CH = 64
_SL = _CH + 8
_MAXB = 4
_BPOS = _MAXB * _CH
_IDSL = (_MAXB - 1) * _CH + _SL
_CE = 128


def _sc_mesh():
    return plsc.VectorSubcoreMesh(
        core_axis_name="c", subcore_axis_name="s",
        num_cores=_NC, num_subcores=_NS)


def _make_bag(N, D, NPT, OB):
    npad = _NW * NPT

    @functools.partial(
        pl.kernel,
        out_type=jax.ShapeDtypeStruct((npad, D), jnp.float32),
        mesh=_sc_mesh(),
        scratch_types=[
            pltpu.VMEM((OB,), jnp.int32),
            pltpu.VMEM((_IDSL,), jnp.int32),
            pltpu.VMEM((_IDSL,), jnp.int32),
            pltpu.VMEM((_MAXB * _SL, D), jnp.float32),
            pltpu.VMEM((_MAXB * _SL, D), jnp.float32),
            pltpu.VMEM((NPT, D), jnp.float32),
            pltpu.VMEM((D,), jnp.float32),
            pltpu.SemaphoreType.DMA,
            pltpu.SemaphoreType.DMA,
            pltpu.SemaphoreType.DMA,
            pltpu.SemaphoreType.DMA,
        ],
    )
    def bag(ids_hbm, off_hbm, emb_hbm, bias_hbm, h_hbm,
            offv, idsA, idsB, rowsA, rowsB, outv, biasv,
            siA, siB, sgA, sgB):
        wid = lax.axis_index("s") * _NC + lax.axis_index("c")
        n0 = pl.multiple_of(wid * NPT, 8)
        pltpu.sync_copy(off_hbm.at[pl.ds(n0, OB)], offv)
        pltpu.sync_copy(bias_hbm, biasv)
        n_end = n0 + jnp.minimum(N - n0, NPT)

        def offs(t):
            op = offv[pl.ds(t - n0, 16)]
            return op[0], op[1]

        def issue_ids(t, base_pos, idsX, siX):
            a = pl.multiple_of((base_pos // 8) * 8, 8)
            pltpu.async_copy(ids_hbm.at[pl.ds(a, _IDSL)], idsX, siX)

        def wait_ids(idsX, siX):
            pltpu.make_async_copy(
                ids_hbm.at[pl.ds(0, _IDSL)], idsX, siX).wait()

        def nbatch(rem):
            return jnp.minimum((rem + _CH - 1) // _CH, _MAXB)

        def fire_gathers(nb, idsX, rowsX, sgX):
            for c in range(_MAXB):
                @pl.when(c < nb)
                def _(c=c):
                    pltpu.async_copy(
                        emb_hbm.at[idsX.at[pl.ds(c * _CH, _SL)]],
                        rowsX.at[pl.ds(c * _SL, _SL), :], sgX)

        def drain_gathers(nb, idsX, rowsX, sgX):
            for c in range(_MAXB):
                @pl.when(c < nb)
                def _(c=c):
                    pltpu.make_async_copy(
                        emb_hbm.at[idsX.at[pl.ds(c * _CH, _SL)]],
                        rowsX.at[pl.ds(c * _SL, _SL), :], sgX).wait()

        def acc_batch(rem, ofs0, rowsX, acc):
            nb = nbatch(rem)

            def chunk_body(c, a8):
                r0 = c * _SL + ofs0
                valid = jnp.clip(rem - c * _CH, 0, _CH)

                def row(r, a2):
                    return tuple(a2[u] + rowsX[r, pl.ds(u * 16, 16)]
                                 for u in range(8))

                return lax.fori_loop(r0, r0 + valid, row, a8)

            return lax.fori_loop(0, nb, chunk_body, acc)

        zero = jnp.zeros((16,), jnp.float32)

        def half(i, idsX, rowsX, siX, sgX, idsY, rowsY, siY, sgY):
            nxt = i + 1

            @pl.when(nxt < n_end)
            def _():
                wait_ids(idsY, siY)
                j0n, j1n = offs(nxt)
                fire_gathers(nbatch(j1n - j0n), idsY, rowsY, sgY)

            @pl.when(i < n_end)
            def _():
                j0, j1 = offs(i)
                k = j1 - j0
                drain_gathers(nbatch(k), idsX, rowsX, sgX)
                ofs0 = j0 - (j0 // 8) * 8
                acc = acc_batch(k, ofs0, rowsX, (zero,) * 8)

                def of_body(m, a8):
                    jm = j0 + m * _BPOS
                    rem = k - m * _BPOS
                    issue_ids(i, jm, idsX, siX)
                    wait_ids(idsX, siX)
                    nb = nbatch(rem)
                    fire_gathers(nb, idsX, rowsX, sgX)
                    drain_gathers(nb, idsX, rowsX, sgX)
                    ofm = jm - (jm // 8) * 8
                    return acc_batch(rem, ofm, rowsX, a8)

                n_extra = jnp.maximum((k - 1) // _BPOS, 0)
                fin = lax.fori_loop(1, 1 + n_extra, of_body, acc)
                for u in range(8):
                    outv[i - n0, pl.ds(u * 16, 16)] = jnp.maximum(
                        fin[u] + biasv[pl.ds(u * 16, 16)], 0.0)

            @pl.when(i + 2 < n_end)
            def _():
                j0n2, _unused = offs(i + 2)
                issue_ids(i + 2, j0n2, idsX, siX)

        @pl.when(n0 < n_end)
        def _():
            j0, j1 = offs(n0)
            issue_ids(n0, j0, idsA, siA)
            wait_ids(idsA, siA)
            fire_gathers(nbatch(j1 - j0), idsA, rowsA, sgA)

        @pl.when(n0 + 1 < n_end)
        def _():
            j0b, _unused = offs(n0 + 1)
            issue_ids(n0 + 1, j0b, idsB, siB)

        def pair(i2, carry):
            i = n0 + 2 * i2
            half(i, idsA, rowsA, siA, sgA, idsB, rowsB, siB, sgB)
            half(i + 1, idsB, rowsB, siB, sgB, idsA, rowsA, siA, sgA)
            return carry

        lax.fori_loop(0, NPT // 2, pair, 0)
        pltpu.sync_copy(outv, h_hbm.at[pl.ds(n0, NPT), :])

    return bag


def _make_edge(N, E, D):
    ept = E // _NS
    nch = ept // _CE
    fpt = ((N // _NS + 127) // 128) * 128
    ncop = fpt // _CE
    npad = _NS * fpt

    @functools.partial(
        pl.kernel,
        out_type=jax.ShapeDtypeStruct((2, npad, D), jnp.float32),
        mesh=_sc_mesh(),
        scratch_types=[
            pltpu.VMEM((_CE,), jnp.int32),
            pltpu.VMEM((_CE,), jnp.int32),
            pltpu.VMEM((_CE,), jnp.int32),
            pltpu.VMEM((_CE,), jnp.int32),
            pltpu.VMEM((_CE + 16,), jnp.float32),
            pltpu.VMEM((_CE + 16,), jnp.float32),
            pltpu.VMEM((_CE, D), jnp.float32),
            pltpu.VMEM((_CE, D), jnp.float32),
            pltpu.VMEM_SHARED((npad, D), jnp.float32),
            pltpu.SemaphoreType.DMA,
            pltpu.SemaphoreType.DMA,
            pltpu.SemaphoreType.DMA,
            pltpu.SemaphoreType.DMA,
        ],
    )
    def edge(src_hbm, dst_hbm, ew_hbm, h_hbm, o_hbm,
             srcA, srcB, dstA, dstB, wA, wB, rowsA, rowsB, acc,
             siA, siB, sgA, sgB):
        cid = lax.axis_index("c")
        sid = lax.axis_index("s")

        zero = jnp.zeros((16,), jnp.float32)

        def zrow(r, carry):
            for u in range(8):
                rowsA[r, pl.ds(u * 16, 16)] = zero
            return carry

        lax.fori_loop(0, _CE, zrow, 0)
        for q in range(ncop):
            pltpu.sync_copy(
                rowsA,
                acc.at[pl.ds(pl.multiple_of(sid * fpt + q * _CE, 8),
                             _CE), :])
        plsc.subcore_barrier()

        e0 = sid * ept

        def idx_copies(go, c, srcX, dstX, wX, siX):
            eb = e0 + c * _CE
            f = pltpu.async_copy if go else pltpu.make_async_copy
            d1 = f(src_hbm.at[pl.ds(eb, _CE)], srcX, siX)
            d2 = f(dst_hbm.at[pl.ds(eb, _CE)], dstX, siX)
            d3 = f(ew_hbm.at[pl.ds(pl.multiple_of(cid * E + eb, 128),
                                   _CE)],
                   wX.at[pl.ds(0, _CE)], siX)
            if not go:
                d1.wait()
                d2.wait()
                d3.wait()

        def fire_gather(srcX, rowsX, sgX):
            pltpu.async_copy(h_hbm.at[srcX], rowsX, sgX)

        def wait_gather(srcX, rowsX, sgX):
            pltpu.make_async_copy(h_hbm.at[srcX], rowsX, sgX).wait()

        def process(rowsX, wX, dstX):
            def g(gi, carry):
                wvec = wX[pl.ds(gi * 16, 16)]
                for i in range(16):
                    e = gi * 16 + i
                    w = wvec[i]
                    for u in range(8):
                        rowsX[e, pl.ds(u * 16, 16)] = (
                            rowsX[e, pl.ds(u * 16, 16)] * w)
                return carry

            lax.fori_loop(0, _CE // 16, g, 0)
            pltpu.sync_copy(rowsX, acc.at[dstX], add=True)

        def half(c, srcX, dstX, wX, rowsX, siX, sgX,
                 srcY, dstY, wY, rowsY, siY, sgY):
            @pl.when(c + 1 < nch)
            def _():
                idx_copies(False, c + 1, srcY, dstY, wY, siY)
                fire_gather(srcY, rowsY, sgY)

            wait_gather(srcX, rowsX, sgX)
            process(rowsX, wX, dstX)

            @pl.when(c + 2 < nch)
            def _():
                idx_copies(True, c + 2, srcX, dstX, wX, siX)

        idx_copies(True, 0, srcA, dstA, wA, siA)
        idx_copies(False, 0, srcA, dstA, wA, siA)
        fire_gather(srcA, rowsA, sgA)
        idx_copies(True, 1, srcB, dstB, wB, siB)

        def pair(c2, carry):
            c = 2 * c2
            half(c, srcA, dstA, wA, rowsA, siA, sgA,
                 srcB, dstB, wB, rowsB, siB, sgB)
            half(c + 1, srcB, dstB, wB, rowsB, siB, sgB,
                 srcA, dstA, wA, rowsA, siA, sgA)
            return carry

        lax.fori_loop(0, nch // 2, pair, 0)
        plsc.subcore_barrier()
        f0 = pl.multiple_of(sid * fpt, 8)
        pltpu.sync_copy(acc.at[pl.ds(f0, fpt), :],
                        o_hbm.at[cid, pl.ds(f0, fpt), :])

    return edge


def _dense(res, ppi, W, b):
    N, D = ppi.shape
    BN = 1000

    def body(res_ref, ppi_ref, w_ref, b_ref, out_ref):
        y = lax.dot_general(ppi_ref[...], w_ref[...],
                            (((1,), (1,)), ((), ())),
                            preferred_element_type=jnp.float32)
        out_ref[...] = jnp.maximum(y + b_ref[...] + res_ref[...], 0.0)

    return pl.pallas_call(
        body,
        grid=(N // BN,),
        in_specs=[
            pl.BlockSpec((BN, D), lambda i: (i, 0)),
            pl.BlockSpec((BN, D), lambda i: (i, 0)),
            pl.BlockSpec((D, D), lambda i: (0, 0)),
            pl.BlockSpec((1, D), lambda i: (0, 0)),
        ],
        out_specs=pl.BlockSpec((BN, D), lambda i: (i, 0)),
        out_shape=jax.ShapeDtypeStruct((N, D), jnp.float32),
    )(res, ppi, W, b.reshape(1, D))


def _proj(h, W_out, b_out):
    N, D = h.shape
    C = W_out.shape[0]
    BN = 1000

    def body(h_ref, w_ref, b_ref, out_ref):
        y = lax.dot_general(h_ref[...], w_ref[...],
                            (((1,), (1,)), ((), ())),
                            preferred_element_type=jnp.float32)
        out_ref[...] = y + b_ref[...]

    return pl.pallas_call(
        body,
        grid=(N // BN,),
        in_specs=[
            pl.BlockSpec((BN, D), lambda i: (i, 0)),
            pl.BlockSpec((C, D), lambda i: (0, 0)),
            pl.BlockSpec((1, C), lambda i: (0, 0)),
        ],
        out_specs=pl.BlockSpec((BN, C), lambda i: (i, 0)),
        out_shape=jax.ShapeDtypeStruct((N, C), jnp.float32),
    )(h, W_out, b_out.reshape(1, C))


def kernel(input_ids, offsets, edge_index, edge_ppi, edge_self,
           emb_table, input_bias, W1, b1, W2, b2, W_out, b_out):
    L = input_ids.shape[0]
    N = offsets.shape[0] - 1
    V, D = emb_table.shape
    E = edge_index.shape[1]

    npt = ((-(-N // _NW) + 7) // 8) * 8
    ob = ((npt + 32) // 8) * 8

    ids_pad = jnp.pad(input_ids.astype(jnp.int32), (0, _IDSL + 8))
    off_pad = jnp.pad(offsets.astype(jnp.int32), (0, ob),
                      constant_values=L)

    h = _make_bag(N, D, npt, ob)(ids_pad, off_pad, emb_table,
                                 input_bias)[:N]

    ep = -(-E // (_NS * 2 * _CE)) * _NS * 2 * _CE
    pad_e = ep - E
    src = jnp.pad(edge_index[0].astype(jnp.int32), (0, pad_e))
    dst = jnp.pad(edge_index[1].astype(jnp.int32), (0, pad_e),
                  constant_values=N)
    ew = jnp.concatenate([jnp.pad(edge_self, (0, pad_e)),
                          jnp.pad(edge_ppi, (0, pad_e))])

    edge_fn = _make_edge(N, ep, D)
    for W, b in ((W1, b1), (W2, b2)):
        o = edge_fn(src, dst, ew, h)
        h = _dense(o[0, :N], o[1, :N], W, b)

    return _proj(h, W_out, b_out)


# speedup vs baseline: 71.4496x; 1.1625x over previous
import functools

import jax
import jax.numpy as jnp
from jax import lax
from jax.experimental import pallas as pl
from jax.experimental.pallas import tpu as pltpu
from jax.experimental.pallas import tpu_sc as plsc

_NC = 2
_NS = 16
_NW = _NC * _NS

_CH = 64
_SL = _CH + 8
_MAXB = 4
_BPOS = _MAXB * _CH
_IDSL = (_MAXB - 1) * _CH + _SL
_CE = 96


def _sc_mesh():
    return plsc.VectorSubcoreMesh(
        core_axis_name="c", subcore_axis_name="s",
        num_cores=_NC, num_subcores=_NS)


def _make_bag(N, D, NPT, OB):
    npad = _NW * NPT

    @functools.partial(
        pl.kernel,
        out_type=jax.ShapeDtypeStruct((npad, D), jnp.float32),
        mesh=_sc_mesh(),
        scratch_types=[
            pltpu.VMEM((OB,), jnp.int32),
            pltpu.VMEM((_IDSL,), jnp.int32),
            pltpu.VMEM((_IDSL,), jnp.int32),
            pltpu.VMEM((_MAXB * _SL, D), jnp.float32),
            pltpu.VMEM((_MAXB * _SL, D), jnp.float32),
            pltpu.VMEM((NPT, D), jnp.float32),
            pltpu.VMEM((D,), jnp.float32),
            pltpu.SemaphoreType.DMA,
            pltpu.SemaphoreType.DMA,
            pltpu.SemaphoreType.DMA,
            pltpu.SemaphoreType.DMA,
        ],
    )
    def bag(ids_hbm, off_hbm, emb_hbm, bias_hbm, h_hbm,
            offv, idsA, idsB, rowsA, rowsB, outv, biasv,
            siA, siB, sgA, sgB):
        wid = lax.axis_index("s") * _NC + lax.axis_index("c")
        n0 = pl.multiple_of(wid * NPT, 8)
        pltpu.sync_copy(off_hbm.at[pl.ds(n0, OB)], offv)
        pltpu.sync_copy(bias_hbm, biasv)
        n_end = n0 + jnp.minimum(N - n0, NPT)

        def offs(t):
            op = offv[pl.ds(t - n0, 16)]
            return op[0], op[1]

        def issue_ids(t, base_pos, idsX, siX):
            a = pl.multiple_of((base_pos // 8) * 8, 8)
            pltpu.async_copy(ids_hbm.at[pl.ds(a, _IDSL)], idsX, siX)

        def wait_ids(idsX, siX):
            pltpu.make_async_copy(
                ids_hbm.at[pl.ds(0, _IDSL)], idsX, siX).wait()

        def nbatch(rem):
            return jnp.minimum((rem + _CH - 1) // _CH, _MAXB)

        def fire_gathers(nb, idsX, rowsX, sgX):
            for c in range(_MAXB):
                @pl.when(c < nb)
                def _(c=c):
                    pltpu.async_copy(
                        emb_hbm.at[idsX.at[pl.ds(c * _CH, _SL)]],
                        rowsX.at[pl.ds(c * _SL, _SL), :], sgX)

        def drain_gathers(nb, idsX, rowsX, sgX):
            for c in range(_MAXB):
                @pl.when(c < nb)
                def _(c=c):
                    pltpu.make_async_copy(
                        emb_hbm.at[idsX.at[pl.ds(c * _CH, _SL)]],
                        rowsX.at[pl.ds(c * _SL, _SL), :], sgX).wait()

        def acc_batch(rem, ofs0, rowsX, acc):
            nb = nbatch(rem)

            def chunk_body(c, a8):
                r0 = c * _SL + ofs0
                valid = jnp.clip(rem - c * _CH, 0, _CH)

                def row(r, a2):
                    return tuple(a2[u] + rowsX[r, pl.ds(u * 16, 16)]
                                 for u in range(8))

                return lax.fori_loop(r0, r0 + valid, row, a8)

            return lax.fori_loop(0, nb, chunk_body, acc)

        zero = jnp.zeros((16,), jnp.float32)

        def half(i, idsX, rowsX, siX, sgX, idsY, rowsY, siY, sgY):
            nxt = i + 1

            @pl.when(nxt < n_end)
            def _():
                wait_ids(idsY, siY)
                j0n, j1n = offs(nxt)
                fire_gathers(nbatch(j1n - j0n), idsY, rowsY, sgY)

            @pl.when(i < n_end)
            def _():
                j0, j1 = offs(i)
                k = j1 - j0
                drain_gathers(nbatch(k), idsX, rowsX, sgX)
                ofs0 = j0 - (j0 // 8) * 8
                acc = acc_batch(k, ofs0, rowsX, (zero,) * 8)

                def of_body(m, a8):
                    jm = j0 + m * _BPOS
                    rem = k - m * _BPOS
                    issue_ids(i, jm, idsX, siX)
                    wait_ids(idsX, siX)
                    nb = nbatch(rem)
                    fire_gathers(nb, idsX, rowsX, sgX)
                    drain_gathers(nb, idsX, rowsX, sgX)
                    ofm = jm - (jm // 8) * 8
                    return acc_batch(rem, ofm, rowsX, a8)

                n_extra = jnp.maximum((k - 1) // _BPOS, 0)
                fin = lax.fori_loop(1, 1 + n_extra, of_body, acc)
                for u in range(8):
                    outv[i - n0, pl.ds(u * 16, 16)] = jnp.maximum(
                        fin[u] + biasv[pl.ds(u * 16, 16)], 0.0)

            @pl.when(i + 2 < n_end)
            def _():
                j0n2, _unused = offs(i + 2)
                issue_ids(i + 2, j0n2, idsX, siX)

        @pl.when(n0 < n_end)
        def _():
            j0, j1 = offs(n0)
            issue_ids(n0, j0, idsA, siA)
            wait_ids(idsA, siA)
            fire_gathers(nbatch(j1 - j0), idsA, rowsA, sgA)

        @pl.when(n0 + 1 < n_end)
        def _():
            j0b, _unused = offs(n0 + 1)
            issue_ids(n0 + 1, j0b, idsB, siB)

        def pair(i2, carry):
            i = n0 + 2 * i2
            half(i, idsA, rowsA, siA, sgA, idsB, rowsB, siB, sgB)
            half(i + 1, idsB, rowsB, siB, sgB, idsA, rowsA, siA, sgA)
            return carry

        lax.fori_loop(0, NPT // 2, pair, 0)
        pltpu.sync_copy(outv, h_hbm.at[pl.ds(n0, NPT), :])

    return bag


def _make_edge(N, E, D):
    ept = E // _NS
    nch = ept // _CE
    fpt = ((N // _NS + 127) // 128) * 128
    ncop = fpt // _CE
    npad = _NS * fpt

    @functools.partial(
        pl.kernel,
        out_type=jax.ShapeDtypeStruct((2, npad, D), jnp.float32),
        mesh=_sc_mesh(),
        scratch_types=(
            [pltpu.VMEM((_CE,), jnp.int32)] * 3
            + [pltpu.VMEM((_CE,), jnp.int32)] * 3
            + [pltpu.VMEM((_CE + 16,), jnp.float32)] * 3
            + [pltpu.VMEM((_CE, D), jnp.float32)] * 3
            + [pltpu.VMEM_SHARED((npad, D), jnp.float32)]
            + [pltpu.SemaphoreType.DMA] * 9
        ),
    )
    def edge(src_hbm, dst_hbm, ew_hbm, h_hbm, o_hbm,
             srcA, srcB, srcC, dstA, dstB, dstC, wA, wB, wC,
             rowsA, rowsB, rowsC, acc,
             siA, siB, siC, sgA, sgB, sgC, ssA, ssB, ssC):
        cid = lax.axis_index("c")
        sid = lax.axis_index("s")

        srcb = (srcA, srcB, srcC)
        dstb = (dstA, dstB, dstC)
        wb = (wA, wB, wC)
        rowsb = (rowsA, rowsB, rowsC)
        sib = (siA, siB, siC)
        sgb = (sgA, sgB, sgC)
        ssb = (ssA, ssB, ssC)

        zero = jnp.zeros((16,), jnp.float32)

        def zrow(r, carry):
            for u in range(8):
                rowsA[r, pl.ds(u * 16, 16)] = zero
            return carry

        lax.fori_loop(0, _CE, zrow, 0)
        for q in range(ncop):
            pltpu.sync_copy(
                rowsA,
                acc.at[pl.ds(pl.multiple_of(sid * fpt + q * _CE, 8),
                             _CE), :])
        zrem = fpt - ncop * _CE
        if zrem:
            pltpu.sync_copy(
                rowsA.at[pl.ds(0, zrem), :],
                acc.at[pl.ds(pl.multiple_of(sid * fpt + ncop * _CE, 8),
                             zrem), :])
        plsc.subcore_barrier()

        e0 = sid * ept

        def idx_copies(go, c, x):
            eb = e0 + c * _CE
            f = pltpu.async_copy if go else pltpu.make_async_copy
            d1 = f(src_hbm.at[pl.ds(eb, _CE)], srcb[x], sib[x])
            d2 = f(dst_hbm.at[pl.ds(eb, _CE)], dstb[x], sib[x])
            d3 = f(ew_hbm.at[pl.ds(pl.multiple_of(cid * E + eb, 96),
                                   _CE)],
                   wb[x].at[pl.ds(0, _CE)], sib[x])
            if not go:
                d1.wait()
                d2.wait()
                d3.wait()

        def fire_gather(x):
            pltpu.async_copy(h_hbm.at[srcb[x]], rowsb[x], sgb[x])

        def wait_gather(x):
            pltpu.make_async_copy(h_hbm.at[srcb[x]], rowsb[x],
                                  sgb[x]).wait()

        def fire_scatter(x):
            pltpu.async_copy(rowsb[x], acc.at[dstb[x]], ssb[x],
                             add=True)

        def wait_scatter(x):
            pltpu.make_async_copy(rowsb[x], acc.at[dstb[x]],
                                  ssb[x]).wait()

        def scale(x):
            rowsX = rowsb[x]
            wX = wb[x]

            def g(gi, carry):
                wvec = wX[pl.ds(gi * 16, 16)]
                for i in range(16):
                    e = gi * 16 + i
                    w = wvec[i]
                    for u in range(8):
                        rowsX[e, pl.ds(u * 16, 16)] = (
                            rowsX[e, pl.ds(u * 16, 16)] * w)
                return carry

            lax.fori_loop(0, _CE // 16, g, 0)

        def stage(c, x, y, z):
            @pl.when(c + 1 < nch)
            def _():
                idx_copies(False, c + 1, y)
                fire_gather(y)

            wait_gather(x)

            @pl.when(c >= 1)
            def _():
                wait_scatter(z)

            @pl.when(c + 2 < nch)
            def _():
                idx_copies(True, c + 2, z)

            scale(x)
            fire_scatter(x)

        idx_copies(True, 0, 0)
        idx_copies(False, 0, 0)
        fire_gather(0)
        idx_copies(True, 1, 1)

        def triple(c3, carry):
            c = 3 * c3
            stage(c, 0, 1, 2)
            stage(c + 1, 1, 2, 0)
            stage(c + 2, 2, 0, 1)
            return carry

        lax.fori_loop(0, nch // 3, triple, 0)
        wait_scatter((nch - 1) % 3)
        plsc.subcore_barrier()
        f0 = pl.multiple_of(sid * fpt, 8)
        pltpu.sync_copy(acc.at[pl.ds(f0, fpt), :],
                        o_hbm.at[cid, pl.ds(f0, fpt), :])

    return edge


def _dense(res, ppi, W, b):
    N, D = ppi.shape
    BN = 1000

    def body(res_ref, ppi_ref, w_ref, b_ref, out_ref):
        y = lax.dot_general(ppi_ref[...], w_ref[...],
                            (((1,), (1,)), ((), ())),
                            preferred_element_type=jnp.float32)
        out_ref[...] = jnp.maximum(y + b_ref[...] + res_ref[...], 0.0)

    return pl.pallas_call(
        body,
        grid=(N // BN,),
        in_specs=[
            pl.BlockSpec((BN, D), lambda i: (i, 0)),
            pl.BlockSpec((BN, D), lambda i: (i, 0)),
            pl.BlockSpec((D, D), lambda i: (0, 0)),
            pl.BlockSpec((1, D), lambda i: (0, 0)),
        ],
        out_specs=pl.BlockSpec((BN, D), lambda i: (i, 0)),
        out_shape=jax.ShapeDtypeStruct((N, D), jnp.float32),
    )(res, ppi, W, b.reshape(1, D))


def _proj(h, W_out, b_out):
    N, D = h.shape
    C = W_out.shape[0]
    BN = 1000

    def body(h_ref, w_ref, b_ref, out_ref):
        y = lax.dot_general(h_ref[...], w_ref[...],
                            (((1,), (1,)), ((), ())),
                            preferred_element_type=jnp.float32)
        out_ref[...] = y + b_ref[...]

    return pl.pallas_call(
        body,
        grid=(N // BN,),
        in_specs=[
            pl.BlockSpec((BN, D), lambda i: (i, 0)),
            pl.BlockSpec((C, D), lambda i: (0, 0)),
            pl.BlockSpec((1, C), lambda i: (0, 0)),
        ],
        out_specs=pl.BlockSpec((BN, C), lambda i: (i, 0)),
        out_shape=jax.ShapeDtypeStruct((N, C), jnp.float32),
    )(h, W_out, b_out.reshape(1, C))


def kernel(input_ids, offsets, edge_index, edge_ppi, edge_self,
           emb_table, input_bias, W1, b1, W2, b2, W_out, b_out):
    L = input_ids.shape[0]
    N = offsets.shape[0] - 1
    V, D = emb_table.shape
    E = edge_index.shape[1]

    npt = ((-(-N // _NW) + 7) // 8) * 8
    ob = ((npt + 32) // 8) * 8

    ids_pad = jnp.pad(input_ids.astype(jnp.int32), (0, _IDSL + 8))
    off_pad = jnp.pad(offsets.astype(jnp.int32), (0, ob),
                      constant_values=L)

    h = _make_bag(N, D, npt, ob)(ids_pad, off_pad, emb_table,
                                 input_bias)[:N]

    ep = -(-E // (_NS * 3 * _CE)) * _NS * 3 * _CE
    pad_e = ep - E
    src = jnp.pad(edge_index[0].astype(jnp.int32), (0, pad_e))
    dst = jnp.pad(edge_index[1].astype(jnp.int32), (0, pad_e),
                  constant_values=N)
    ew = jnp.concatenate([jnp.pad(edge_self, (0, pad_e)),
                          jnp.pad(edge_ppi, (0, pad_e))])

    edge_fn = _make_edge(N, ep, D)
    for W, b in ((W1, b1), (W2, b2)):
        o = edge_fn(src, dst, ew, h)
        h = _dense(o[0, :N], o[1, :N], W, b)

    return _proj(h, W_out, b_out)
